# baseline (speedup 1.0000x reference)
import functools

import jax
import jax.numpy as jnp
from jax import lax
from jax.experimental import pallas as pl
from jax.experimental.pallas import tpu as pltpu
from jax.experimental.pallas import tpu_sc as plsc

BATCH = 16384
DIM = 32
LANES = 16

_info = plsc.get_sparse_core_info()
_NC = _info.num_cores
_NS = _info.num_subcores
_NW = _NC * _NS
_BPW = BATCH // _NW
_CHUNK = 128
_NCHUNK = _BPW // _CHUNK

_mesh = plsc.VectorSubcoreMesh(core_axis_name="c", subcore_axis_name="s")


@functools.partial(
    pl.kernel,
    mesh=_mesh,
    compiler_params=pltpu.CompilerParams(use_tc_tiling_on_sc=False),
    out_type=jax.ShapeDtypeStruct((BATCH, DIM), jnp.float32),
    scratch_types=[
        pltpu.VMEM((_NCHUNK, _CHUNK), jnp.int32),
        pltpu.VMEM((_NCHUNK, _CHUNK), jnp.int32),
        pltpu.VMEM((_BPW, DIM), jnp.float32),
        pltpu.VMEM((_BPW, DIM), jnp.float32),
        pltpu.SemaphoreType.DMA,
    ],
)
def _mf_sc(uidx_hbm, iidx_hbm, u_hbm, i_hbm, out_hbm,
           uidx_v, iidx_v, urows, irows, sem):
    wid = lax.axis_index("s") * _NC + lax.axis_index("c")
    base = wid * _BPW

    for j in range(_NCHUNK):
        src = pl.ds(base + j * _CHUNK, _CHUNK)
        pltpu.sync_copy(uidx_hbm.at[src], uidx_v.at[j])
        pltpu.sync_copy(iidx_hbm.at[src], iidx_v.at[j])

    copies = []
    for j in range(_NCHUNK):
        dst = pl.ds(j * _CHUNK, _CHUNK)
        copies.append(pltpu.async_copy(u_hbm.at[uidx_v.at[j]],
                                       urows.at[dst], sem))
        copies.append(pltpu.async_copy(i_hbm.at[iidx_v.at[j]],
                                       irows.at[dst], sem))
    for c in copies:
        c.wait()

    def body(r, carry):
        for c in range(DIM // LANES):
            sl = pl.ds(c * LANES, LANES)
            urows[r, sl] = urows[r, sl] * irows[r, sl]
        return carry

    lax.fori_loop(0, _BPW, body, 0)

    pltpu.sync_copy(urows, out_hbm.at[pl.ds(base, _BPW)])


def kernel(userIdx, itemIdx, uEmbd, iEmbd):
    return _mf_sc(userIdx, itemIdx, uEmbd, iEmbd)


# speedup vs baseline: 1.8277x; 1.8277x over previous
import functools

import jax
import jax.numpy as jnp
from jax import lax
from jax.experimental import pallas as pl
from jax.experimental.pallas import tpu as pltpu
from jax.experimental.pallas import tpu_sc as plsc

BATCH = 16384
DIM = 32
LANES = 16
TILE = 128

_info = plsc.get_sparse_core_info()
_NC = _info.num_cores
_NS = _info.num_subcores
_NW = _NC * _NS
_BPW = BATCH // _NW
_NG = _BPW // LANES

_mesh = plsc.VectorSubcoreMesh(core_axis_name="c", subcore_axis_name="s")


@functools.partial(
    pl.kernel,
    mesh=_mesh,
    compiler_params=pltpu.CompilerParams(needs_layout_passes=False),
    out_type=jax.ShapeDtypeStruct((DIM, BATCH), jnp.float32),
    scratch_types=[
        pltpu.VMEM((_BPW,), jnp.int32),
        pltpu.VMEM((_BPW,), jnp.int32),
        pltpu.VMEM((DIM, TILE), jnp.float32),
        pltpu.VMEM((DIM, TILE), jnp.float32),
        pltpu.VMEM((DIM, _BPW), jnp.float32),
        pltpu.SemaphoreType.DMA,
        pltpu.SemaphoreType.DMA,
    ],
)
def _mf_sc(uidx_hbm, iidx_hbm, ut_hbm, it_hbm, out_hbm,
           uidx_v, iidx_v, ubuf, ibuf, ocols, semu, semi):
    wid = lax.axis_index("s") * _NC + lax.axis_index("c")
    base = wid * _BPW

    pltpu.sync_copy(uidx_hbm.at[pl.ds(base, _BPW)], uidx_v)
    pltpu.sync_copy(iidx_hbm.at[pl.ds(base, _BPW)], iidx_v)

    rows_lo = lax.iota(jnp.int32, LANES)
    rows_hi = rows_lo + LANES

    def group(g, carry):
        uvec = uidx_v[pl.ds(g * LANES, LANES)]
        ivec = iidx_v[pl.ds(g * LANES, LANES)]
        for l in range(LANES):
            ui = uvec[l]
            ii = ivec[l]
            b = g * LANES + l
            off_u = pl.multiple_of((ui >> 7) << 7, TILE)
            off_i = pl.multiple_of((ii >> 7) << 7, TILE)
            cu = pltpu.async_copy(ut_hbm.at[:, pl.ds(off_u, TILE)], ubuf,
                                  semu)
            ci = pltpu.async_copy(it_hbm.at[:, pl.ds(off_i, TILE)], ibuf,
                                  semi)
            cu.wait()
            ci.wait()
            cu_vec = jnp.full((LANES,), ui & 127, dtype=jnp.int32)
            ci_vec = jnp.full((LANES,), ii & 127, dtype=jnp.int32)
            b_vec = jnp.full((LANES,), b, dtype=jnp.int32)
            for rows in (rows_lo, rows_hi):
                uv = plsc.load_gather(ubuf, [rows, cu_vec])
                iv = plsc.load_gather(ibuf, [rows, ci_vec])
                plsc.store_scatter(ocols, [rows, b_vec], uv * iv)
        return carry

    lax.fori_loop(0, _NG, group, 0)

    pltpu.sync_copy(ocols, out_hbm.at[:, pl.ds(base, _BPW)])


def kernel(userIdx, itemIdx, uEmbd, iEmbd):
    out_t = _mf_sc(userIdx, itemIdx, uEmbd.T, iEmbd.T)
    return out_t.T


# speedup vs baseline: 3.8832x; 2.1247x over previous
import functools

import jax
import jax.numpy as jnp
from jax import lax
from jax.experimental import pallas as pl
from jax.experimental.pallas import tpu as pltpu
from jax.experimental.pallas import tpu_sc as plsc

BATCH = 16384
DIM = 32
LANES = 16
TILE = 128
DEPTH = 4

_info = plsc.get_sparse_core_info()
_NC = _info.num_cores
_NS = _info.num_subcores
_NW = _NC * _NS
_BPW = BATCH // _NW
_NG = _BPW // LANES

_mesh = plsc.VectorSubcoreMesh(core_axis_name="c", subcore_axis_name="s")


@functools.partial(
    pl.kernel,
    mesh=_mesh,
    compiler_params=pltpu.CompilerParams(needs_layout_passes=False),
    out_type=jax.ShapeDtypeStruct((DIM, BATCH), jnp.float32),
    scratch_types=[
        pltpu.VMEM((_BPW,), jnp.int32),
        pltpu.VMEM((_BPW,), jnp.int32),
        pltpu.VMEM((DEPTH, DIM, TILE), jnp.float32),
        pltpu.VMEM((DEPTH, DIM, TILE), jnp.float32),
        pltpu.VMEM((DIM, _BPW), jnp.float32),
        [pltpu.SemaphoreType.DMA] * DEPTH,
        [pltpu.SemaphoreType.DMA] * DEPTH,
    ],
)
def _mf_sc(uidx_hbm, iidx_hbm, ut_hbm, it_hbm, out_hbm,
           uidx_v, iidx_v, ubufs, ibufs, ocols, semus, semis):
    wid = lax.axis_index("s") * _NC + lax.axis_index("c")
    base = wid * _BPW

    pltpu.sync_copy(uidx_hbm.at[pl.ds(base, _BPW)], uidx_v)
    pltpu.sync_copy(iidx_hbm.at[pl.ds(base, _BPW)], iidx_v)

    rows_lo = lax.iota(jnp.int32, LANES)
    rows_hi = rows_lo + LANES

    def fire(uvec, ivec, l):
        s = l % DEPTH
        off_u = pl.multiple_of((uvec[l] >> 7) << 7, TILE)
        off_i = pl.multiple_of((ivec[l] >> 7) << 7, TILE)
        pltpu.async_copy(ut_hbm.at[:, pl.ds(off_u, TILE)], ubufs.at[s],
                         semus[s])
        pltpu.async_copy(it_hbm.at[:, pl.ds(off_i, TILE)], ibufs.at[s],
                         semis[s])

    def drain_and_use(uvec, ivec, l, b):
        s = l % DEPTH
        pltpu.make_async_copy(ut_hbm.at[:, pl.ds(0, TILE)], ubufs.at[s],
                              semus[s]).wait()
        pltpu.make_async_copy(it_hbm.at[:, pl.ds(0, TILE)], ibufs.at[s],
                              semis[s]).wait()
        cu_vec = jnp.full((LANES,), uvec[l] & 127, dtype=jnp.int32)
        ci_vec = jnp.full((LANES,), ivec[l] & 127, dtype=jnp.int32)
        b_vec = jnp.full((LANES,), b, dtype=jnp.int32)
        for rows in (rows_lo, rows_hi):
            uv = plsc.load_gather(ubufs.at[s], [rows, cu_vec])
            iv = plsc.load_gather(ibufs.at[s], [rows, ci_vec])
            plsc.store_scatter(ocols, [rows, b_vec], uv * iv)

    uvec0 = uidx_v[pl.ds(0, LANES)]
    ivec0 = iidx_v[pl.ds(0, LANES)]
    for l in range(DEPTH):
        fire(uvec0, ivec0, l)

    def group(g, carry):
        uvec = uidx_v[pl.ds(g * LANES, LANES)]
        ivec = iidx_v[pl.ds(g * LANES, LANES)]
        uvec_n = uidx_v[pl.ds(g * LANES + LANES, LANES)]
        ivec_n = iidx_v[pl.ds(g * LANES + LANES, LANES)]
        for l in range(LANES):
            drain_and_use(uvec, ivec, l, g * LANES + l)
            if l < LANES - DEPTH:
                fire(uvec, ivec, l + DEPTH)
            else:
                fire(uvec_n, ivec_n, l + DEPTH - LANES)
        return carry

    lax.fori_loop(0, _NG - 1, group, 0)

    uvec = uidx_v[pl.ds((_NG - 1) * LANES, LANES)]
    ivec = iidx_v[pl.ds((_NG - 1) * LANES, LANES)]
    for l in range(LANES):
        drain_and_use(uvec, ivec, l, (_NG - 1) * LANES + l)
        if l < LANES - DEPTH:
            fire(uvec, ivec, l + DEPTH)

    pltpu.sync_copy(ocols, out_hbm.at[:, pl.ds(base, _BPW)])


def kernel(userIdx, itemIdx, uEmbd, iEmbd):
    out_t = _mf_sc(userIdx, itemIdx, uEmbd.T, iEmbd.T)
    return out_t.T


# speedup vs baseline: 3.9329x; 1.0128x over previous
import functools

import jax
import jax.numpy as jnp
from jax import lax
from jax.experimental import pallas as pl
from jax.experimental.pallas import tpu as pltpu
from jax.experimental.pallas import tpu_sc as plsc

BATCH = 16384
DIM = 32
LANES = 16
TILE = 128
DEPTH = 8

_info = plsc.get_sparse_core_info()
_NC = _info.num_cores
_NS = _info.num_subcores
_NW = _NC * _NS
_BPW = BATCH // _NW
_NG = _BPW // LANES

_mesh = plsc.VectorSubcoreMesh(core_axis_name="c", subcore_axis_name="s")


@functools.partial(
    pl.kernel,
    mesh=_mesh,
    compiler_params=pltpu.CompilerParams(needs_layout_passes=False),
    out_type=jax.ShapeDtypeStruct((DIM, BATCH), jnp.float32),
    scratch_types=[
        pltpu.VMEM((_BPW,), jnp.int32),
        pltpu.VMEM((_BPW,), jnp.int32),
        pltpu.VMEM((DEPTH, DIM, TILE), jnp.float32),
        pltpu.VMEM((DEPTH, DIM, TILE), jnp.float32),
        pltpu.VMEM((DIM, _BPW), jnp.float32),
        [pltpu.SemaphoreType.DMA] * DEPTH,
        [pltpu.SemaphoreType.DMA] * DEPTH,
    ],
)
def _mf_sc(uidx_hbm, iidx_hbm, ut_hbm, it_hbm, out_hbm,
           uidx_v, iidx_v, ubufs, ibufs, ocols, semus, semis):
    wid = lax.axis_index("s") * _NC + lax.axis_index("c")
    base = wid * _BPW

    pltpu.sync_copy(uidx_hbm.at[pl.ds(base, _BPW)], uidx_v)
    pltpu.sync_copy(iidx_hbm.at[pl.ds(base, _BPW)], iidx_v)

    rows_lo = lax.iota(jnp.int32, LANES)
    rows_hi = rows_lo + LANES

    def fire(uvec, ivec, l):
        s = l % DEPTH
        off_u = pl.multiple_of((uvec[l] >> 7) << 7, TILE)
        off_i = pl.multiple_of((ivec[l] >> 7) << 7, TILE)
        pltpu.async_copy(ut_hbm.at[:, pl.ds(off_u, TILE)], ubufs.at[s],
                         semus[s])
        pltpu.async_copy(it_hbm.at[:, pl.ds(off_i, TILE)], ibufs.at[s],
                         semis[s])

    def drain_and_use(uvec, ivec, l, b):
        s = l % DEPTH
        pltpu.make_async_copy(ut_hbm.at[:, pl.ds(0, TILE)], ubufs.at[s],
                              semus[s]).wait()
        pltpu.make_async_copy(it_hbm.at[:, pl.ds(0, TILE)], ibufs.at[s],
                              semis[s]).wait()
        cu_vec = jnp.full((LANES,), uvec[l] & 127, dtype=jnp.int32)
        ci_vec = jnp.full((LANES,), ivec[l] & 127, dtype=jnp.int32)
        b_vec = jnp.full((LANES,), b, dtype=jnp.int32)
        for rows in (rows_lo, rows_hi):
            uv = plsc.load_gather(ubufs.at[s], [rows, cu_vec])
            iv = plsc.load_gather(ibufs.at[s], [rows, ci_vec])
            plsc.store_scatter(ocols, [rows, b_vec], uv * iv)

    uvec0 = uidx_v[pl.ds(0, LANES)]
    ivec0 = iidx_v[pl.ds(0, LANES)]
    for l in range(DEPTH):
        fire(uvec0, ivec0, l)

    def group(g, carry):
        uvec = uidx_v[pl.ds(g * LANES, LANES)]
        ivec = iidx_v[pl.ds(g * LANES, LANES)]
        uvec_n = uidx_v[pl.ds(g * LANES + LANES, LANES)]
        ivec_n = iidx_v[pl.ds(g * LANES + LANES, LANES)]
        for l in range(LANES):
            drain_and_use(uvec, ivec, l, g * LANES + l)
            if l < LANES - DEPTH:
                fire(uvec, ivec, l + DEPTH)
            else:
                fire(uvec_n, ivec_n, l + DEPTH - LANES)
        return carry

    lax.fori_loop(0, _NG - 1, group, 0)

    uvec = uidx_v[pl.ds((_NG - 1) * LANES, LANES)]
    ivec = iidx_v[pl.ds((_NG - 1) * LANES, LANES)]
    for l in range(LANES):
        drain_and_use(uvec, ivec, l, (_NG - 1) * LANES + l)
        if l < LANES - DEPTH:
            fire(uvec, ivec, l + DEPTH)

    pltpu.sync_copy(ocols, out_hbm.at[:, pl.ds(base, _BPW)])


def kernel(userIdx, itemIdx, uEmbd, iEmbd):
    out_t = _mf_sc(userIdx, itemIdx, uEmbd.T, iEmbd.T)
    return out_t.T
